# Initial kernel scaffold; baseline (speedup 1.0000x reference)
import jax
import jax.numpy as jnp
from jax.experimental import pallas as pl


def kernel(batch, adj, W1, b1, W2, b2):
    raise NotImplementedError("write your pallas kernel here")


# speedup vs baseline: 115.9417x; 115.9417x over previous
import jax
import jax.numpy as jnp
from jax.experimental import pallas as pl


def _gcn_kernel(x_ref, adj_ref, w1_ref, b1_ref, w2_ref, b2_ref, out_ref,
                *, B, Nn):
    BN = B * Nn
    f32 = jnp.float32

    x = x_ref[...]
    adj = adj_ref[...]
    w1 = w1_ref[...]
    b1 = b1_ref[...]
    w2 = w2_ref[...]
    b2 = b2_ref[...]

    colsum = jnp.sum(adj, axis=0, keepdims=True)
    deg = f32(B) * colsum + 1.0
    dis = jnp.where(deg > 0, jax.lax.rsqrt(deg), 0.0)

    xw = jnp.dot(x, w1, preferred_element_type=f32)

    p_mod = jax.lax.broadcasted_iota(jnp.int32, (BN, Nn), 0) % Nn
    b_idx = jax.lax.broadcasted_iota(jnp.int32, (BN, Nn), 1)
    C1 = (p_mod == b_idx).astype(f32)
    a_idx = jax.lax.broadcasted_iota(jnp.int32, (Nn, BN), 0)
    q_mod = jax.lax.broadcasted_iota(jnp.int32, (Nn, BN), 1) % Nn
    C2 = (a_idx == q_mod).astype(f32)

    t1 = jax.lax.dot_general(C1, adj, (((1,), (1,)), ((), ())),
                             preferred_element_type=f32)
    TA = jnp.dot(t1, C2, preferred_element_type=f32)

    dis_p = jax.lax.dot_general(C1, dis, (((1,), (1,)), ((), ())),
                                preferred_element_type=f32)
    dis_q = jnp.dot(dis, C2, preferred_element_type=f32)

    rp = jax.lax.broadcasted_iota(jnp.int32, (BN, BN), 0)
    cq = jax.lax.broadcasted_iota(jnp.int32, (BN, BN), 1)
    same_node = ((rp % Nn) == (cq % Nn)).astype(f32)
    same_graph = ((rp // Nn) == (cq // Nn)).astype(f32)

    BD = same_graph * (dis_p * (f32(B) * TA + same_node) * dis_q)

    agg = jnp.dot(BD, xw, preferred_element_type=f32)
    h = jnp.maximum(agg + b1, 0.0)

    bi = jax.lax.broadcasted_iota(jnp.int32, (B, BN), 0)
    pj = jax.lax.broadcasted_iota(jnp.int32, (B, BN), 1) // Nn
    P = (bi == pj).astype(f32) * (1.0 / f32(Nn))
    pooled = jnp.dot(P, h, preferred_element_type=f32)

    out_ref[...] = jnp.dot(pooled, w2, preferred_element_type=f32) + b2


def kernel(batch, adj, W1, b1, W2, b2):
    B, Nn, F = batch.shape
    H = W1.shape[1]
    C = W2.shape[1]
    x = batch.reshape(B * Nn, F)
    import functools
    body = functools.partial(_gcn_kernel, B=B, Nn=Nn)
    out = pl.pallas_call(
        body,
        out_shape=jax.ShapeDtypeStruct((B, C), batch.dtype),
    )(x, adj, W1, b1.reshape(1, H), W2, b2.reshape(1, C))
    return out
